# baseline (speedup 1.0000x reference)
import math

import jax
import jax.numpy as jnp
from jax.experimental import pallas as pl
from jax.experimental.pallas import tpu as pltpu

B, S = 2, 2048
HIDDEN = 768
NUM_GROUPS = 12
PER_HEAD = 12
SHORT = HIDDEN // PER_HEAD
NH = 12
HD = HIDDEN // NH
SCALEUP = 1.0 / PER_HEAD

_TS = 128
_RT = 512
_RSTEPS = S // _RT


def _router_body(x_ref, w1_ref, b1_ref, w2_ref, b2_ref, o_ref, acc_ref):
    i = pl.program_id(0)

    @pl.when(i == 0)
    def _init():
        acc_ref[...] = jnp.zeros_like(acc_ref)

    xs = x_ref[...].reshape(B, _RT, NH, SHORT)
    acc_ref[...] += jnp.sum(xs, axis=1).reshape(B * NH, SHORT)

    @pl.when(i == _RSTEPS - 1)
    def _finish():
        m = acc_ref[...] * (1.0 / S)
        h1 = jnp.dot(m, w1_ref[...], preferred_element_type=jnp.float32)
        h1 = h1 + b1_ref[...]
        g = 0.5 * h1 * (1.0 + jax.lax.erf(h1 * (1.0 / math.sqrt(2.0))))
        h2 = jnp.dot(g, w2_ref[...], preferred_element_type=jnp.float32)
        h2 = h2 + b2_ref[...]
        o_ref[...] = jax.nn.softmax(h2, axis=-1).reshape(B, NH, NUM_GROUPS)


_NI = S // _TS
_NBUF = 3
_NQ = 4
_HPQ = NH // _NQ


def _context_body(a_hbm, v_ref, o_ref, abuf, sems):
    step = pl.program_id(0) * _NI + pl.program_id(1)
    nsteps = B * _NI

    def _copies(s):
        bb = s // _NI
        ii = s % _NI
        slot = jax.lax.rem(s, _NBUF)
        return [
            pltpu.make_async_copy(
                a_hbm.at[bb, pl.ds(q * _HPQ, _HPQ), pl.ds(ii * _TS, _TS), :],
                abuf.at[slot, pl.ds(q * _HPQ, _HPQ)],
                sems.at[slot, q],
            )
            for q in range(_NQ)
        ]

    @pl.when(step == 0)
    def _prologue():
        for s in range(_NBUF):
            for c in _copies(s):
                c.start()

    for c in _copies(step):
        c.wait()

    slot = jax.lax.rem(step, _NBUF)
    accs = []
    for h in range(NH):
        accs.append(jnp.dot(abuf[slot, h], v_ref[0, h],
                            preferred_element_type=jnp.float32))
    o_ref[0] = jnp.concatenate(accs, axis=-1) * SCALEUP

    @pl.when(step + _NBUF < nsteps)
    def _prefetch():
        for c in _copies(step + _NBUF):
            c.start()


@jax.jit
def kernel(input_data_seq, attention_probs, value_layer, W1, b1, W2, b2):
    x3 = input_data_seq.reshape(B, S * NH, SHORT)
    prob = pl.pallas_call(
        _router_body,
        grid=(_RSTEPS,),
        in_specs=[
            pl.BlockSpec((B, _RT * NH, SHORT), lambda i: (0, i, 0)),
            pl.BlockSpec((SHORT, NUM_GROUPS), lambda i: (0, 0)),
            pl.BlockSpec((1, NUM_GROUPS), lambda i: (0, 0)),
            pl.BlockSpec((NUM_GROUPS, NUM_GROUPS), lambda i: (0, 0)),
            pl.BlockSpec((1, NUM_GROUPS), lambda i: (0, 0)),
        ],
        out_specs=pl.BlockSpec((B, NH, NUM_GROUPS), lambda i: (0, 0, 0)),
        out_shape=jax.ShapeDtypeStruct((B, NH, NUM_GROUPS), jnp.float32),
        scratch_shapes=[pltpu.VMEM((B * NH, SHORT), jnp.float32)],
        compiler_params=pltpu.CompilerParams(
            dimension_semantics=("arbitrary",),
        ),
    )(x3, W1, b1.reshape(1, NUM_GROUPS), W2, b2.reshape(1, NUM_GROUPS))

    grid = (B, S // _TS)
    out = pl.pallas_call(
        _context_body,
        grid=grid,
        in_specs=[
            pl.BlockSpec(memory_space=pl.ANY),
            pl.BlockSpec((1, NH, S, HD), lambda b, i: (b, 0, 0, 0),
                         pipeline_mode=pl.Buffered(buffer_count=1)),
        ],
        out_specs=pl.BlockSpec((1, _TS, HIDDEN), lambda b, i: (b, i, 0)),
        out_shape=jax.ShapeDtypeStruct((B, S, HIDDEN), jnp.float32),
        scratch_shapes=[
            pltpu.VMEM((_NBUF, NH, _TS, S), jnp.float32),
            pltpu.SemaphoreType.DMA((_NBUF, _NQ)),
        ],
        compiler_params=pltpu.CompilerParams(
            dimension_semantics=("arbitrary", "arbitrary"),
        ),
    )(attention_probs, value_layer)
    context = out.reshape(B, S, NH, HD)

    return (prob, context, value_layer)


# speedup vs baseline: 1.0227x; 1.0227x over previous
import math

import jax
import jax.numpy as jnp
from jax.experimental import pallas as pl
from jax.experimental.pallas import tpu as pltpu

B, S = 2, 2048
HIDDEN = 768
NUM_GROUPS = 12
PER_HEAD = 12
SHORT = HIDDEN // PER_HEAD
NH = 12
HD = HIDDEN // NH
SCALEUP = 1.0 / PER_HEAD

_TS = 128
_RT = 512
_RSTEPS = S // _RT


def _router_body(x_ref, w1_ref, b1_ref, w2_ref, b2_ref, ones_ref, o_ref,
                 acc_ref):
    i = pl.program_id(0)

    @pl.when(i == 0)
    def _init():
        acc_ref[...] = jnp.zeros_like(acc_ref)

    acc_ref[...] += jnp.sum(x_ref[...], axis=1)

    @pl.when(i == _RSTEPS - 1)
    def _finish():
        m = acc_ref[...] * (1.0 / S)
        h1 = jnp.dot(m, w1_ref[...], preferred_element_type=jnp.float32)
        h1 = h1 + b1_ref[...]
        g = 0.5 * h1 * (1.0 + jax.lax.erf(h1 * (1.0 / math.sqrt(2.0))))
        h2 = jnp.dot(g, w2_ref[...], preferred_element_type=jnp.float32)
        h2 = h2 + b2_ref[...]
        e = jnp.exp(h2 - jnp.max(h2, axis=-1, keepdims=True))
        denom = jnp.dot(e, ones_ref[...], preferred_element_type=jnp.float32)
        o_ref[...] = e / denom


_NI = S // _TS
_NBUF = 3
_NQ = 4
_HPQ = NH // _NQ


def _context_body(a_hbm, v_ref, o_ref, abuf, sems):
    step = pl.program_id(0) * _NI + pl.program_id(1)
    nsteps = B * _NI

    def _copies(s):
        bb = s // _NI
        ii = s % _NI
        slot = jax.lax.rem(s, _NBUF)
        return [
            pltpu.make_async_copy(
                a_hbm.at[bb, pl.ds(q * _HPQ, _HPQ), pl.ds(ii * _TS, _TS), :],
                abuf.at[slot, pl.ds(q * _HPQ, _HPQ)],
                sems.at[slot, q],
            )
            for q in range(_NQ)
        ]

    @pl.when(step == 0)
    def _prologue():
        for s in range(_NBUF):
            for c in _copies(s):
                c.start()

    for c in _copies(step):
        c.wait()

    slot = jax.lax.rem(step, _NBUF)
    for h in range(NH):
        acc = jnp.dot(abuf[slot, h], v_ref[0, h],
                      preferred_element_type=jnp.float32)
        o_ref[0, :, h, :] = acc * SCALEUP

    @pl.when(step + _NBUF < nsteps)
    def _prefetch():
        for c in _copies(step + _NBUF):
            c.start()


@jax.jit
def kernel(input_data_seq, attention_probs, value_layer, W1, b1, W2, b2):
    NG = NH * NUM_GROUPS
    eye = jnp.eye(NH, dtype=jnp.float32)
    w1bd = (eye[:, None, :, None] * W1[None, :, None, :]).reshape(HIDDEN, NG)
    w2bd = (eye[:, None, :, None] * W2[None, :, None, :]).reshape(NG, NG)
    onesbd = (eye[:, None, :, None]
              * jnp.ones((NUM_GROUPS, NUM_GROUPS), jnp.float32)[None, :, None, :]
              ).reshape(NG, NG)
    b1t = jnp.tile(b1, NH).reshape(1, NG)
    b2t = jnp.tile(b2, NH).reshape(1, NG)

    pflat = pl.pallas_call(
        _router_body,
        grid=(_RSTEPS,),
        in_specs=[
            pl.BlockSpec((B, _RT, HIDDEN), lambda i: (0, i, 0)),
            pl.BlockSpec((HIDDEN, NG), lambda i: (0, 0)),
            pl.BlockSpec((1, NG), lambda i: (0, 0)),
            pl.BlockSpec((NG, NG), lambda i: (0, 0)),
            pl.BlockSpec((1, NG), lambda i: (0, 0)),
            pl.BlockSpec((NG, NG), lambda i: (0, 0)),
        ],
        out_specs=pl.BlockSpec((B, NG), lambda i: (0, 0)),
        out_shape=jax.ShapeDtypeStruct((B, NG), jnp.float32),
        scratch_shapes=[pltpu.VMEM((B, HIDDEN), jnp.float32)],
        compiler_params=pltpu.CompilerParams(
            dimension_semantics=("arbitrary",),
        ),
    )(input_data_seq, w1bd, b1t, w2bd, b2t, onesbd)
    prob = pflat.reshape(B, NH, NUM_GROUPS)

    grid = (B, S // _TS)
    out = pl.pallas_call(
        _context_body,
        grid=grid,
        in_specs=[
            pl.BlockSpec(memory_space=pl.ANY),
            pl.BlockSpec((1, NH, S, HD), lambda b, i: (b, 0, 0, 0),
                         pipeline_mode=pl.Buffered(buffer_count=1)),
        ],
        out_specs=pl.BlockSpec((1, _TS, NH, HD), lambda b, i: (b, i, 0, 0)),
        out_shape=jax.ShapeDtypeStruct((B, S, NH, HD), jnp.float32),
        scratch_shapes=[
            pltpu.VMEM((_NBUF, NH, _TS, S), jnp.float32),
            pltpu.SemaphoreType.DMA((_NBUF, _NQ)),
        ],
        compiler_params=pltpu.CompilerParams(
            dimension_semantics=("arbitrary", "arbitrary"),
        ),
    )(attention_probs, value_layer)

    return (prob, out, value_layer)
